# baseline (speedup 1.0000x reference)
import functools

import jax
import jax.numpy as jnp
from jax import lax
from jax.experimental import pallas as pl
from jax.experimental.pallas import tpu as pltpu
from jax.experimental.pallas import tpu_sc as plsc

_NC = 2
_NS = 16
_NW = _NC * _NS
_L = 16

_B = 16384
_D = 64
_BPW = _B // _NW
_NCHUNK = _BPW // _L


@functools.partial(
    pl.kernel,
    out_type=jax.ShapeDtypeStruct((_B, _D), jnp.float32),
    mesh=plsc.VectorSubcoreMesh(core_axis_name="c", subcore_axis_name="s"),
    scratch_types=[
        pltpu.VMEM((_BPW,), jnp.int32),
        pltpu.VMEM((_BPW, _D), jnp.float32),
        pltpu.SemaphoreType.DMA,
    ],
)
def _embed_lookup(labels_hbm, table_hbm, out_hbm, idx_v, out_v, sem):
    wid = lax.axis_index("s") * _NC + lax.axis_index("c")
    base = wid * _BPW
    pltpu.sync_copy(labels_hbm.at[pl.ds(base, _BPW)], idx_v)

    def chunk_body(c, carry):
        chunk = idx_v[pl.ds(c * _L, _L)]
        for l in range(_L):
            i = chunk[l]
            pltpu.async_copy(
                table_hbm.at[pl.ds(i, 1), :],
                out_v.at[pl.ds(c * _L + l, 1), :],
                sem,
            )
        return carry

    lax.fori_loop(0, _NCHUNK, chunk_body, 0)
    pltpu.make_async_copy(
        table_hbm.at[pl.ds(0, _BPW), :], out_v, sem
    ).wait()
    pltpu.sync_copy(out_v, out_hbm.at[pl.ds(base, _BPW)])


def kernel(labels, table):
    return _embed_lookup(labels.astype(jnp.int32), table)


# speedup vs baseline: 1.7748x; 1.7748x over previous
import functools

import jax
import jax.numpy as jnp
from jax import lax
from jax.experimental import pallas as pl
from jax.experimental.pallas import tpu as pltpu
from jax.experimental.pallas import tpu_sc as plsc

_NC = 2
_NS = 16
_NW = _NC * _NS
_L = 16

_B = 16384
_D = 64
_BPW = _B // _NW
_FW = 128
_RING = 8
_STEPS = _BPW // _RING


@functools.partial(
    pl.kernel,
    out_type=jax.ShapeDtypeStruct((_D, _B), jnp.float32),
    mesh=plsc.VectorSubcoreMesh(core_axis_name="c", subcore_axis_name="s"),
    scratch_types=[
        pltpu.VMEM((_BPW,), jnp.int32),
        pltpu.VMEM((_RING, _D, _FW), jnp.float32),
        pltpu.VMEM((_D, _BPW), jnp.float32),
        pltpu.SemaphoreType.DMA,
        pltpu.SemaphoreType.DMA,
        pltpu.SemaphoreType.DMA,
        pltpu.SemaphoreType.DMA,
        pltpu.SemaphoreType.DMA,
        pltpu.SemaphoreType.DMA,
        pltpu.SemaphoreType.DMA,
        pltpu.SemaphoreType.DMA,
    ],
    compiler_params=pltpu.CompilerParams(needs_layout_passes=False),
)
def _embed_lookup(labels_hbm, table_t_hbm, out_t_hbm, idx_v, ring_v, out_v,
                  *sems):
    wid = lax.axis_index("s") * _NC + lax.axis_index("c")
    base = wid * _BPW
    pltpu.sync_copy(labels_hbm.at[pl.ds(base, _BPW)], idx_v)

    rows = lax.broadcasted_iota(jnp.int32, (_L,), 0)
    lanes = rows

    def splat_idx(k):
        return plsc.load_gather(idx_v, [jnp.full((_L,), k, jnp.int32)])

    def to_scalar(v):
        return jnp.sum(jnp.where(lanes == 0, v, 0))

    def fetch(k, r):
        i = to_scalar(splat_idx(k))
        off = pl.multiple_of((i // _FW) * _FW, _FW)
        pltpu.async_copy(
            table_t_hbm.at[:, pl.ds(off, _FW)], ring_v.at[r], sems[r]
        )

    def wait_extract(k, r):
        pltpu.make_async_copy(
            table_t_hbm.at[:, pl.ds(0, _FW)], ring_v.at[r], sems[r]
        ).wait()
        lane = splat_idx(k) % _FW
        col = jnp.full((_L,), k, jnp.int32)
        block = ring_v.at[r]
        for q in range(_D // _L):
            vals = plsc.load_gather(block, [rows + q * _L, lane])
            plsc.store_scatter(out_v, [rows + q * _L, col], vals)

    for r in range(_RING):
        fetch(r, r)

    def step_body(s, carry):
        for r in range(_RING):
            k = s * _RING + r
            wait_extract(k, r)

            @pl.when(k + _RING < _BPW)
            def _():
                fetch(k + _RING, r)
        return carry

    lax.fori_loop(0, _STEPS, step_body, 0)
    pltpu.sync_copy(out_v, out_t_hbm.at[:, pl.ds(base, _BPW)])


def kernel(labels, table):
    out_t = _embed_lookup(labels.astype(jnp.int32), table.T)
    return out_t.T


# speedup vs baseline: 1.7773x; 1.0014x over previous
import functools

import jax
import jax.numpy as jnp
from jax import lax
from jax.experimental import pallas as pl
from jax.experimental.pallas import tpu as pltpu
from jax.experimental.pallas import tpu_sc as plsc

_NC = 2
_NS = 16
_NW = _NC * _NS
_L = 16

_B = 16384
_D = 64
_BPW = _B // _NW
_FW = 128
_RING = 8
_NHALF = _BPW // _RING


@functools.partial(
    pl.kernel,
    out_type=jax.ShapeDtypeStruct((_D, _B), jnp.float32),
    mesh=plsc.VectorSubcoreMesh(core_axis_name="c", subcore_axis_name="s"),
    scratch_types=[
        pltpu.VMEM((_BPW + _L,), jnp.int32),
        pltpu.VMEM((_RING, _D, _FW), jnp.float32),
        pltpu.VMEM((_D, _BPW), jnp.float32),
        pltpu.SemaphoreType.DMA,
        pltpu.SemaphoreType.DMA,
        pltpu.SemaphoreType.DMA,
        pltpu.SemaphoreType.DMA,
        pltpu.SemaphoreType.DMA,
        pltpu.SemaphoreType.DMA,
        pltpu.SemaphoreType.DMA,
        pltpu.SemaphoreType.DMA,
    ],
    compiler_params=pltpu.CompilerParams(needs_layout_passes=False),
)
def _embed_lookup(labels_hbm, table_t_hbm, out_t_hbm, idx_v, ring_v, out_v,
                  *sems):
    wid = lax.axis_index("s") * _NC + lax.axis_index("c")
    base = wid * _BPW
    pltpu.sync_copy(labels_hbm.at[pl.ds(base, _BPW)], idx_v.at[pl.ds(0, _BPW)])

    rows = [lax.broadcasted_iota(jnp.int32, (_L,), 0) + q * _L
            for q in range(_D // _L)]

    def fetch(i, r):
        off = pl.multiple_of((i // _FW) * _FW, _FW)
        pltpu.async_copy(
            table_t_hbm.at[:, pl.ds(off, _FW)], ring_v.at[r], sems[r]
        )

    def wait_extract(i, k, r):
        pltpu.make_async_copy(
            table_t_hbm.at[:, pl.ds(0, _FW)], ring_v.at[r], sems[r]
        ).wait()
        lane = jnp.full((_L,), i % _FW, jnp.int32)
        col = jnp.full((_L,), k, jnp.int32)
        block = ring_v.at[r]
        for q in range(_D // _L):
            vals = plsc.load_gather(block, [rows[q], lane])
            plsc.store_scatter(out_v, [rows[q], col], vals)

    cur0 = idx_v[pl.ds(0, _L)]
    for r in range(_RING):
        fetch(cur0[r], r)

    def half_body(h, cur):
        nxt = idx_v[pl.ds((h + 1) * _RING, _L)]
        for r in range(_RING):
            wait_extract(cur[r], h * _RING + r, r)

            @pl.when(h + 1 < _NHALF)
            def _():
                fetch(nxt[r], r)
        return nxt

    lax.fori_loop(0, _NHALF, half_body, cur0)
    pltpu.sync_copy(out_v, out_t_hbm.at[:, pl.ds(base, _BPW)])


def kernel(labels, table):
    out_t = _embed_lookup(labels.astype(jnp.int32), table.T)
    return out_t.T
